# baseline (speedup 1.0000x reference)
import functools

import jax
import jax.numpy as jnp
from jax import lax
from jax.experimental import pallas as pl
from jax.experimental.pallas import tpu as pltpu
from jax.experimental.pallas import tpu_sc as plsc

SEQ = 200
DIM = 128
BATCH = 1024
ROWS = BATCH * SEQ
IROW = 128
CHUNK = 128
NCHUNK = ROWS // CHUNK

_info = plsc.get_sparse_core_info()
_NC, _NS = _info.num_cores, _info.num_subcores
_NW = _NC * _NS
_CPW = NCHUNK // _NW
_NBUF = 6
_STAGE = 40


def _emb_kernel(idx_hbm, token_hbm, pos_hbm, out_hbm,
                idx_v, rows_v, pos_sh, psem, gsem, wsem):
    sub = lax.axis_index("s")
    wid = sub * _NC + lax.axis_index("c")
    c0 = wid * _CPW
    pltpu.sync_copy(idx_hbm.at[wid], idx_v)

    @pl.when(sub < (3 * SEQ) // _STAGE)
    def _stage_pos():
        d0 = pl.multiple_of(sub * _STAGE, 8)
        s0 = pl.multiple_of(lax.rem(sub * _STAGE, SEQ), 8)
        pltpu.sync_copy(pos_hbm.at[pl.ds(s0, _STAGE)],
                        pos_sh.at[pl.ds(d0, _STAGE)])

    plsc.subcore_barrier()

    def prefill(kk, b):
        bm = pl.multiple_of(lax.rem((c0 + kk) * CHUNK, SEQ), 8)
        pltpu.async_copy(pos_sh.at[pl.ds(bm, CHUNK)], rows_v.at[b],
                         psem.at[b])

    def gather(kk, b):
        pltpu.make_async_copy(pos_sh.at[pl.ds(0, CHUNK)], rows_v.at[b],
                              psem.at[b]).wait()
        pltpu.async_copy(token_hbm.at[idx_v.at[kk]], rows_v.at[b],
                         gsem.at[b], add=True)

    def write(kk, b):
        pltpu.make_async_copy(token_hbm.at[idx_v.at[0]], rows_v.at[b],
                              gsem.at[b]).wait()
        pltpu.async_copy(rows_v.at[b], out_hbm.at[pl.ds((c0 + kk) * CHUNK,
                                                        CHUNK)], wsem.at[b])

    prefill(0, 0)
    prefill(1, 1)
    gather(0, 0)

    def body(j, carry):
        b = lax.rem(j, _NBUF)
        b1 = lax.rem(j + 1, _NBUF)
        b2 = lax.rem(j + 2, _NBUF)

        @pl.when(j + 2 < _CPW)
        def _start_prefill():
            @pl.when(j >= _NBUF - 2)
            def _wait_old_write():
                pltpu.make_async_copy(
                    rows_v.at[b2], out_hbm.at[pl.ds(0, CHUNK)], wsem.at[b2]
                ).wait()

            prefill(j + 2, b2)

        write(j, b)

        @pl.when(j + 1 < _CPW)
        def _start_gather():
            gather(j + 1, b1)

        return carry

    lax.fori_loop(0, _CPW, body, 0)
    for b in range(_NBUF):
        pltpu.make_async_copy(rows_v.at[b], out_hbm.at[pl.ds(0, CHUNK)],
                              wsem.at[b]).wait()


@jax.jit
def kernel(inputs, token_table, position_table):
    idx = inputs.astype(jnp.int32).reshape(_NW, _CPW, IROW)
    run = functools.partial(
        pl.kernel,
        mesh=plsc.VectorSubcoreMesh(core_axis_name="c", subcore_axis_name="s"),
        out_type=jax.ShapeDtypeStruct((ROWS, DIM), jnp.float32),
        scratch_types=[
            pltpu.VMEM((_CPW, IROW), jnp.int32),
            pltpu.VMEM((_NBUF, CHUNK, DIM), jnp.float32),
            pltpu.VMEM_SHARED((3 * SEQ, DIM), jnp.float32),
            pltpu.SemaphoreType.DMA((_NBUF,)),
            pltpu.SemaphoreType.DMA((_NBUF,)),
            pltpu.SemaphoreType.DMA((_NBUF,)),
        ],
    )(_emb_kernel)
    out = run(idx, token_table, position_table)
    return out.reshape(BATCH, SEQ, DIM)


# speedup vs baseline: 1.2148x; 1.2148x over previous
import functools

import jax
import jax.numpy as jnp
from jax import lax
from jax.experimental import pallas as pl
from jax.experimental.pallas import tpu as pltpu
from jax.experimental.pallas import tpu_sc as plsc

SEQ = 200
DIM = 128
BATCH = 1024
ROWS = BATCH * SEQ
CHUNK = 128
NCHUNK = ROWS // CHUNK

_info = plsc.get_sparse_core_info()
_NC, _NS = _info.num_cores, _info.num_subcores
_NW = _NC * _NS
_CPW = NCHUNK // _NW
_NBUF = 6
_STAGE = 40


def _emb_kernel(idx_hbm, token_hbm, pos_hbm, out_hbm,
                idx_v, rows_v, pos_sh, psem, gsem, wsem):
    sub = lax.axis_index("s")
    wid = sub * _NC + lax.axis_index("c")
    c0 = wid * _CPW
    pltpu.sync_copy(idx_hbm.at[wid], idx_v)

    @pl.when(sub < (2 * SEQ) // _STAGE)
    def _stage_pos():
        d0 = pl.multiple_of(sub * _STAGE, 8)
        s0 = pl.multiple_of(lax.rem(sub * _STAGE, SEQ), 8)
        pltpu.sync_copy(pos_hbm.at[pl.ds(s0, _STAGE)],
                        pos_sh.at[pl.ds(d0, _STAGE)])

    plsc.subcore_barrier()

    def prefill(kk, b):
        bm = pl.multiple_of(lax.rem((c0 + kk) * CHUNK, SEQ), 8)
        pltpu.async_copy(pos_sh.at[pl.ds(bm, CHUNK)], rows_v.at[b],
                         psem.at[b])

    def gather(kk, b):
        pltpu.make_async_copy(pos_sh.at[pl.ds(0, CHUNK)], rows_v.at[b],
                              psem.at[b]).wait()
        pltpu.async_copy(token_hbm.at[idx_v.at[kk]], rows_v.at[b],
                         gsem.at[b], add=True)

    def write(kk, b):
        pltpu.make_async_copy(token_hbm.at[idx_v.at[0]], rows_v.at[b],
                              gsem.at[b]).wait()
        pltpu.async_copy(rows_v.at[b], out_hbm.at[pl.ds((c0 + kk) * CHUNK,
                                                        CHUNK)], wsem.at[b])

    prefill(0, 0)
    prefill(1, 1)
    gather(0, 0)

    def body(j, carry):
        b = lax.rem(j, _NBUF)
        b1 = lax.rem(j + 1, _NBUF)
        b2 = lax.rem(j + 2, _NBUF)

        @pl.when(j + 2 < _CPW)
        def _start_prefill():
            @pl.when(j >= _NBUF - 2)
            def _wait_old_write():
                pltpu.make_async_copy(
                    rows_v.at[b2], out_hbm.at[pl.ds(0, CHUNK)], wsem.at[b2]
                ).wait()

            prefill(j + 2, b2)

        @pl.when(j + 1 < _CPW)
        def _start_gather():
            gather(j + 1, b1)

        write(j, b)
        return carry

    lax.fori_loop(0, _CPW, body, 0)
    for b in range(_NBUF):
        pltpu.make_async_copy(rows_v.at[b], out_hbm.at[pl.ds(0, CHUNK)],
                              wsem.at[b]).wait()


@jax.jit
def kernel(inputs, token_table, position_table):
    idx = inputs.astype(jnp.int32).reshape(_NW, _CPW, CHUNK)
    run = functools.partial(
        pl.kernel,
        mesh=plsc.VectorSubcoreMesh(core_axis_name="c", subcore_axis_name="s"),
        out_type=jax.ShapeDtypeStruct((ROWS, DIM), jnp.float32),
        scratch_types=[
            pltpu.VMEM((_CPW, CHUNK), jnp.int32),
            pltpu.VMEM((_NBUF, CHUNK, DIM), jnp.float32),
            pltpu.VMEM_SHARED((2 * SEQ, DIM), jnp.float32),
            pltpu.SemaphoreType.DMA((_NBUF,)),
            pltpu.SemaphoreType.DMA((_NBUF,)),
            pltpu.SemaphoreType.DMA((_NBUF,)),
        ],
    )(_emb_kernel)
    out = run(idx, token_table, position_table)
    return out.reshape(BATCH, SEQ, DIM)
